# Initial kernel scaffold; baseline (speedup 1.0000x reference)
import jax
import jax.numpy as jnp
from jax.experimental import pallas as pl


def kernel(input_signal, weight_matrix):
    raise NotImplementedError("write your pallas kernel here")


# speedup vs baseline: 7.3663x; 7.3663x over previous
import jax
import jax.numpy as jnp
from jax.experimental import pallas as pl

MAP_H, MAP_W = 128, 64


def _som_kernel(x_ref, w_ref, out_ref):
    x = x_ref[...]
    w = w_ref[...]
    xw = jax.lax.dot_general(
        x, w, (((1,), (1,)), ((), ())), preferred_element_type=jnp.float32
    )
    w2 = w * w
    ones_d = jnp.ones((1, w.shape[1]), dtype=jnp.float32)
    wn2 = jax.lax.dot_general(
        ones_d, w2, (((1,), (1,)), ((), ())), preferred_element_type=jnp.float32
    )
    xn2 = jnp.sum(x * x, axis=1, keepdims=True)
    d2 = jnp.maximum(wn2 + xn2 - 2.0 * xw, 0.0)
    s = -10.0 * jnp.sqrt(d2)
    m = jnp.max(s, axis=1, keepdims=True)
    e = jnp.exp(s - m)
    out_ref[...] = e / jnp.sum(e, axis=1, keepdims=True)


def kernel(input_signal, weight_matrix):
    b = input_signal.shape[0]
    k = weight_matrix.shape[0]
    out = pl.pallas_call(
        _som_kernel,
        out_shape=jax.ShapeDtypeStruct((b, k), jnp.float32),
    )(input_signal, weight_matrix)
    return out.reshape(b, MAP_H, MAP_W)
